# baseline (speedup 1.0000x reference)
import jax
import jax.numpy as jnp
from jax import lax
from jax.experimental import pallas as pl
from jax.experimental.pallas import tpu as pltpu
from jax.experimental.pallas import tpu_sc as plsc


def _adjacency_sc_kernel(B, Nm, D, Npad):
    info = plsc.get_sparse_core_info()
    NC = info.num_cores
    mesh = plsc.VectorSubcoreMesh(core_axis_name="c", subcore_axis_name="s")
    nchunks = (Nm + 15) // 16

    def body(eye_hbm, et_hbm, out_hbm, e_v, a_v):
        w = lax.axis_index("s") * NC + lax.axis_index("c")
        pltpu.sync_copy(et_hbm.at[w], e_v)
        pltpu.sync_copy(eye_hbm, a_v)
        ones = jnp.full((16,), 1.0, jnp.float32)
        for d in range(D):
            for c in range(nchunks):
                n = c * 16 + lax.iota(jnp.int32, 16)
                cols = e_v[d, pl.ds(c * 16, 16)]
                mask = n < Nm
                plsc.addupdate_scatter(
                    a_v, [jnp.minimum(n, Nm - 1), cols], ones, mask=mask)
        pltpu.sync_copy(a_v, out_hbm.at[w])

    return pl.kernel(
        body,
        mesh=mesh,
        out_type=jax.ShapeDtypeStruct((B, Nm, Nm), jnp.float32),
        scratch_types=[
            pltpu.VMEM((D, Npad), jnp.int32),
            pltpu.VMEM((Nm, Nm), jnp.float32),
        ],
        compiler_params=pltpu.CompilerParams(needs_layout_passes=False),
    )


def _protein_kernel(
    p_atoms_ref,
    p_edges_ref,
    W_p1_ref, b_p1_ref,
    W_p2_ref, b_p2_ref,
    W_gopp_ref, b_gopp_ref,
    fpp_ref,
):
    BB = p_atoms_ref.shape[0]
    f32 = jnp.float32
    b_p1 = b_p1_ref[:].reshape(1, -1)
    b_p2 = b_p2_ref[:].reshape(1, -1)
    b_gopp = b_gopp_ref[:].reshape(1, -1)

    for i in range(BB):
        pa = p_atoms_ref[i][:, :].astype(f32)
        pe = p_edges_ref[i][:, :].astype(f32)
        t1 = jnp.dot(pe, pa, preferred_element_type=f32)
        x1 = jax.nn.relu(jnp.dot(t1, W_p1_ref[:, :], preferred_element_type=f32)
                         + b_p1)
        t2 = jnp.dot(pe, x1, preferred_element_type=f32)
        x2 = jax.nn.relu(jnp.dot(t2, W_p2_ref[:, :], preferred_element_type=f32)
                         + b_p2)
        tp = jnp.tanh(jnp.dot(x2, W_gopp_ref[:, :], preferred_element_type=f32)
                      + b_gopp)
        fpp_ref[0, pl.ds(i, 1), :] = jnp.sum(tp, axis=0, keepdims=True)


def _mol_head_kernel(
    m_atoms_ref,
    m_bonds_ref,
    adj_ref,
    fpp_ref,
    W_m1_ref, b_m1_ref,
    W_m2_ref, b_m2_ref,
    W_gop_ref, b_gop_ref,
    W_fc1_ref, b_fc1_ref,
    W_fc2_ref, b_fc2_ref,
    out_ref,
):
    B, Nm, Fa = m_atoms_ref.shape
    H = W_m1_ref.shape[1]
    f32 = jnp.float32
    hi = jax.lax.Precision.HIGHEST

    W_m1a = W_m1_ref[0:Fa, :]
    W_m1b = W_m1_ref[Fa:Fa + 6, :]
    W_m2a = W_m2_ref[0:H, :]
    W_m2b = W_m2_ref[H:H + 6, :]
    W_m2c = W_m2_ref[H + 6:H + 12, :]
    W_gopa = W_gop_ref[0:H, :]
    W_gopb = W_gop_ref[H:H + 6, :]
    W_fc1a = W_fc1_ref[0:H, :]
    W_fc1b = W_fc1_ref[H:2 * H, :]
    b_m1 = b_m1_ref[:].reshape(1, -1)
    b_m2 = b_m2_ref[:].reshape(1, -1)
    b_gop = b_gop_ref[:].reshape(1, -1)
    b_fc1 = b_fc1_ref[:].reshape(1, -1)
    b_fc2 = b_fc2_ref[:].reshape(1, -1)

    As = [adj_ref[b] for b in range(B)]

    bsum_l, s1_l = [], []
    for b in range(B):
        atoms = m_atoms_ref[b]
        bsum_l.append(jnp.sum(m_bonds_ref[b], axis=1))
        s1_l.append(jnp.dot(As[b], atoms, preferred_element_type=f32,
                            precision=hi))
    bsum = jnp.concatenate(bsum_l, axis=0)
    s1 = jnp.concatenate(s1_l, axis=0)
    hm1 = jax.nn.relu(
        jnp.dot(s1, W_m1a, preferred_element_type=f32)
        + jnp.dot(bsum, W_m1b, preferred_element_type=f32)
        + b_m1)

    s2h_l, s2b_l = [], []
    for b in range(B):
        hm1_b = jax.lax.slice(hm1, (b * Nm, 0), ((b + 1) * Nm, H))
        s2h_l.append(jnp.dot(As[b], hm1_b, preferred_element_type=f32,
                             precision=hi))
        s2b_l.append(jnp.dot(As[b], bsum_l[b], preferred_element_type=f32,
                             precision=hi))
    s2h = jnp.concatenate(s2h_l, axis=0)
    s2b = jnp.concatenate(s2b_l, axis=0)
    hm2 = jax.nn.relu(
        jnp.dot(s2h, W_m2a, preferred_element_type=f32)
        + jnp.dot(s2b, W_m2b, preferred_element_type=f32)
        + jnp.dot(bsum, W_m2c, preferred_element_type=f32)
        + b_m2)

    tm = jnp.tanh(
        jnp.dot(hm2, W_gopa, preferred_element_type=f32)
        + jnp.dot(bsum, W_gopb, preferred_element_type=f32)
        + b_gop)

    rows = jax.lax.broadcasted_iota(jnp.int32, (B, B * Nm), 0)
    cols = jax.lax.broadcasted_iota(jnp.int32, (B, B * Nm), 1)
    S = (cols // Nm == rows).astype(f32)
    fp_m = jnp.dot(S, tm, preferred_element_type=f32, precision=hi)

    fp_p = fpp_ref[:, :]
    inter = jax.nn.sigmoid(
        jnp.dot(fp_m, W_fc1a, preferred_element_type=f32)
        + jnp.dot(fp_p, W_fc1b, preferred_element_type=f32)
        + b_fc1)
    logits = jnp.dot(inter, W_fc2_ref[:, :], preferred_element_type=f32) \
        + b_fc2
    mx = jnp.max(logits, axis=1, keepdims=True)
    ex = jnp.exp(logits - mx)
    probs = ex / jnp.sum(ex, axis=1, keepdims=True)
    out_ref[:, :, :] = probs.reshape(B, 1, 2)


@jax.jit
def kernel(m_atoms, m_bonds, p_atoms, p_edges,
           W_m1, b_m1, W_m2, b_m2, W_p1, b_p1, W_p2, b_p2,
           W_gop, b_gop, W_gopp, b_gopp, W_fc1, b_fc1, W_fc2, b_fc2,
           m_edges):
    B, Nm, _ = m_atoms.shape
    D = m_edges.shape[2]
    m_edges32 = m_edges.astype(jnp.int32)

    Npad = ((Nm + 15) // 16) * 16
    e_t = jnp.pad(m_edges32.transpose(0, 2, 1), ((0, 0), (0, 0), (0, Npad - Nm)))
    eye = jnp.eye(Nm, dtype=jnp.float32)
    adj = _adjacency_sc_kernel(B, Nm, D, Npad)(eye, e_t)

    p_atoms16 = p_atoms.astype(jnp.bfloat16)
    p_edges16 = p_edges.astype(jnp.bfloat16)

    BB = 2

    def whole(x):
        return pl.BlockSpec(x.shape, lambda b: (0,) * x.ndim)

    stream = lambda x: pl.BlockSpec((BB,) + x.shape[1:],
                                    lambda b: (b,) + (0,) * (x.ndim - 1))

    fpp = pl.pallas_call(
        _protein_kernel,
        grid=(B // BB,),
        in_specs=[stream(p_atoms16), stream(p_edges16),
                  whole(W_p1), whole(b_p1), whole(W_p2), whole(b_p2),
                  whole(W_gopp), whole(b_gopp)],
        out_specs=pl.BlockSpec((1, BB, 128), lambda b: (b, 0, 0)),
        out_shape=jax.ShapeDtypeStruct((B // BB, BB, 128), jnp.float32),
        compiler_params=pltpu.CompilerParams(
            dimension_semantics=("arbitrary",)),
    )(p_atoms16, p_edges16, W_p1, b_p1, W_p2, b_p2, W_gopp, b_gopp)
    fpp = fpp.reshape(B, 128)

    mol_operands = [m_atoms, m_bonds, adj, fpp,
                    W_m1, b_m1, W_m2, b_m2,
                    W_gop, b_gop, W_fc1, b_fc1, W_fc2, b_fc2]
    out = pl.pallas_call(
        _mol_head_kernel,
        in_specs=[pl.BlockSpec(x.shape, lambda *_, __nd=x.ndim: (0,) * __nd)
                  for x in mol_operands],
        out_specs=pl.BlockSpec((B, 1, 2), lambda: (0, 0, 0)),
        out_shape=jax.ShapeDtypeStruct((B, 1, 2), jnp.float32),
    )(*mol_operands)
    return out.reshape(B, 2)


# speedup vs baseline: 1.0260x; 1.0260x over previous
import jax
import jax.numpy as jnp
from jax import lax
from jax.experimental import pallas as pl
from jax.experimental.pallas import tpu as pltpu
from jax.experimental.pallas import tpu_sc as plsc


def _adjacency_sc_kernel(B, Nm, D, Npad):
    info = plsc.get_sparse_core_info()
    NC = info.num_cores
    mesh = plsc.VectorSubcoreMesh(core_axis_name="c", subcore_axis_name="s")
    nchunks = (Nm + 15) // 16

    def body(et_hbm, out_hbm, e_v, a_v):
        w = lax.axis_index("s") * NC + lax.axis_index("c")
        pltpu.sync_copy(et_hbm.at[w], e_v)
        ones = jnp.full((16,), 1.0, jnp.float32)
        zeros = jnp.zeros((16,), jnp.float32)
        for r in range(Nm):
            for c in range(Npad // 16):
                a_v[r, pl.ds(c * 16, 16)] = zeros
        for c in range(nchunks):
            n = c * 16 + lax.iota(jnp.int32, 16)
            nc = jnp.minimum(n, Nm - 1)
            plsc.store_scatter(a_v, [nc, nc], ones, mask=n < Nm)
        for d in range(D):
            for c in range(nchunks):
                n = c * 16 + lax.iota(jnp.int32, 16)
                cols = e_v[d, pl.ds(c * 16, 16)]
                mask = n < Nm
                plsc.addupdate_scatter(
                    a_v, [jnp.minimum(n, Nm - 1), cols], ones, mask=mask)
        pltpu.sync_copy(a_v, out_hbm.at[w])

    return pl.kernel(
        body,
        mesh=mesh,
        out_type=jax.ShapeDtypeStruct((B, Nm, Npad), jnp.float32),
        scratch_types=[
            pltpu.VMEM((D, Npad), jnp.int32),
            pltpu.VMEM((Nm, Npad), jnp.float32),
        ],
        compiler_params=pltpu.CompilerParams(needs_layout_passes=False),
    )


def _protein_kernel(
    p_atoms_ref,
    p_edges_ref,
    W_p1_ref, b_p1_ref,
    W_p2_ref, b_p2_ref,
    W_gopp_ref, b_gopp_ref,
    fpp_ref,
):
    BB = p_atoms_ref.shape[0]
    f32 = jnp.float32
    b_p1 = b_p1_ref[:].reshape(1, -1)
    b_p2 = b_p2_ref[:].reshape(1, -1)
    b_gopp = b_gopp_ref[:].reshape(1, -1)

    for i in range(BB):
        pa = p_atoms_ref[i][:, :].astype(f32)
        pe = p_edges_ref[i][:, :].astype(f32)
        t1 = jnp.dot(pe, pa, preferred_element_type=f32)
        x1 = jax.nn.relu(jnp.dot(t1, W_p1_ref[:, :], preferred_element_type=f32)
                         + b_p1)
        t2 = jnp.dot(pe, x1, preferred_element_type=f32)
        x2 = jax.nn.relu(jnp.dot(t2, W_p2_ref[:, :], preferred_element_type=f32)
                         + b_p2)
        tp = jnp.tanh(jnp.dot(x2, W_gopp_ref[:, :], preferred_element_type=f32)
                      + b_gopp)
        fpp_ref[0, pl.ds(i, 1), :] = jnp.sum(tp, axis=0, keepdims=True)


def _mol_head_kernel(
    m_atoms_ref,
    m_bonds_ref,
    adj_ref,
    fpp_ref,
    W_m1_ref, b_m1_ref,
    W_m2_ref, b_m2_ref,
    W_gop_ref, b_gop_ref,
    W_fc1_ref, b_fc1_ref,
    W_fc2_ref, b_fc2_ref,
    out_ref,
):
    B, Nm, Fa = m_atoms_ref.shape
    H = W_m1_ref.shape[1]
    f32 = jnp.float32
    hi = jax.lax.Precision.HIGHEST

    W_m1a = W_m1_ref[0:Fa, :]
    W_m1b = W_m1_ref[Fa:Fa + 6, :]
    W_m2a = W_m2_ref[0:H, :]
    W_m2b = W_m2_ref[H:H + 6, :]
    W_m2c = W_m2_ref[H + 6:H + 12, :]
    W_gopa = W_gop_ref[0:H, :]
    W_gopb = W_gop_ref[H:H + 6, :]
    W_fc1a = W_fc1_ref[0:H, :]
    W_fc1b = W_fc1_ref[H:2 * H, :]
    b_m1 = b_m1_ref[:].reshape(1, -1)
    b_m2 = b_m2_ref[:].reshape(1, -1)
    b_gop = b_gop_ref[:].reshape(1, -1)
    b_fc1 = b_fc1_ref[:].reshape(1, -1)
    b_fc2 = b_fc2_ref[:].reshape(1, -1)

    Nm_ = adj_ref.shape[1]
    As = [adj_ref[b][:, 0:Nm_] for b in range(B)]

    bsum_l, s1_l = [], []
    for b in range(B):
        atoms = m_atoms_ref[b]
        bsum_l.append(jnp.sum(m_bonds_ref[b], axis=1))
        s1_l.append(jnp.dot(As[b], atoms, preferred_element_type=f32,
                            precision=hi))
    bsum = jnp.concatenate(bsum_l, axis=0)
    s1 = jnp.concatenate(s1_l, axis=0)
    hm1 = jax.nn.relu(
        jnp.dot(s1, W_m1a, preferred_element_type=f32)
        + jnp.dot(bsum, W_m1b, preferred_element_type=f32)
        + b_m1)

    s2h_l, s2b_l = [], []
    for b in range(B):
        hm1_b = jax.lax.slice(hm1, (b * Nm, 0), ((b + 1) * Nm, H))
        s2h_l.append(jnp.dot(As[b], hm1_b, preferred_element_type=f32,
                             precision=hi))
        s2b_l.append(jnp.dot(As[b], bsum_l[b], preferred_element_type=f32,
                             precision=hi))
    s2h = jnp.concatenate(s2h_l, axis=0)
    s2b = jnp.concatenate(s2b_l, axis=0)
    hm2 = jax.nn.relu(
        jnp.dot(s2h, W_m2a, preferred_element_type=f32)
        + jnp.dot(s2b, W_m2b, preferred_element_type=f32)
        + jnp.dot(bsum, W_m2c, preferred_element_type=f32)
        + b_m2)

    tm = jnp.tanh(
        jnp.dot(hm2, W_gopa, preferred_element_type=f32)
        + jnp.dot(bsum, W_gopb, preferred_element_type=f32)
        + b_gop)

    rows = jax.lax.broadcasted_iota(jnp.int32, (B, B * Nm), 0)
    cols = jax.lax.broadcasted_iota(jnp.int32, (B, B * Nm), 1)
    S = (cols // Nm == rows).astype(f32)
    fp_m = jnp.dot(S, tm, preferred_element_type=f32, precision=hi)

    fp_p = fpp_ref[:, :]
    inter = jax.nn.sigmoid(
        jnp.dot(fp_m, W_fc1a, preferred_element_type=f32)
        + jnp.dot(fp_p, W_fc1b, preferred_element_type=f32)
        + b_fc1)
    logits = jnp.dot(inter, W_fc2_ref[:, :], preferred_element_type=f32) \
        + b_fc2
    mx = jnp.max(logits, axis=1, keepdims=True)
    ex = jnp.exp(logits - mx)
    probs = ex / jnp.sum(ex, axis=1, keepdims=True)
    out_ref[:, :, :] = probs.reshape(B, 1, 2)


@jax.jit
def kernel(m_atoms, m_bonds, p_atoms, p_edges,
           W_m1, b_m1, W_m2, b_m2, W_p1, b_p1, W_p2, b_p2,
           W_gop, b_gop, W_gopp, b_gopp, W_fc1, b_fc1, W_fc2, b_fc2,
           m_edges):
    B, Nm, _ = m_atoms.shape
    D = m_edges.shape[2]
    m_edges32 = m_edges.astype(jnp.int32)

    Npad = ((Nm + 15) // 16) * 16
    e_t = jnp.pad(m_edges32.transpose(0, 2, 1), ((0, 0), (0, 0), (0, Npad - Nm)))
    adj = _adjacency_sc_kernel(B, Nm, D, Npad)(e_t)

    p_atoms16 = p_atoms.astype(jnp.bfloat16)
    p_edges16 = p_edges.astype(jnp.bfloat16)

    BB = 2

    def whole(x):
        return pl.BlockSpec(x.shape, lambda b: (0,) * x.ndim)

    stream = lambda x: pl.BlockSpec((BB,) + x.shape[1:],
                                    lambda b: (b,) + (0,) * (x.ndim - 1))

    fpp = pl.pallas_call(
        _protein_kernel,
        grid=(B // BB,),
        in_specs=[stream(p_atoms16), stream(p_edges16),
                  whole(W_p1), whole(b_p1), whole(W_p2), whole(b_p2),
                  whole(W_gopp), whole(b_gopp)],
        out_specs=pl.BlockSpec((1, BB, 128), lambda b: (b, 0, 0)),
        out_shape=jax.ShapeDtypeStruct((B // BB, BB, 128), jnp.float32),
        compiler_params=pltpu.CompilerParams(
            dimension_semantics=("arbitrary",)),
    )(p_atoms16, p_edges16, W_p1, b_p1, W_p2, b_p2, W_gopp, b_gopp)
    fpp = fpp.reshape(B, 128)

    mol_operands = [m_atoms, m_bonds, adj, fpp,
                    W_m1, b_m1, W_m2, b_m2,
                    W_gop, b_gop, W_fc1, b_fc1, W_fc2, b_fc2]
    out = pl.pallas_call(
        _mol_head_kernel,
        in_specs=[pl.BlockSpec(x.shape, lambda *_, __nd=x.ndim: (0,) * __nd)
                  for x in mol_operands],
        out_specs=pl.BlockSpec((B, 1, 2), lambda: (0, 0, 0)),
        out_shape=jax.ShapeDtypeStruct((B, 1, 2), jnp.float32),
    )(*mol_operands)
    return out.reshape(B, 2)
